# baseline (speedup 1.0000x reference)
import functools

import jax
import jax.numpy as jnp
from jax import lax
from jax.experimental import pallas as pl
from jax.experimental.pallas import tpu as pltpu
from jax.experimental.pallas import tpu_sc as plsc

VOCAB = 1000000
HIDDEN = 64
BATCH = 16384
HIST = 50

NC = 2
NS = 16
NW = NC * NS
LANES = 16

CHUNK = 128
IPW = BATCH // NW
TPW = IPW // CHUNK
NBUF = 4

PAD = 128
TBLK = 8192


def _tc_table_transpose():
  def body(wt_ref, out_ref):
    out_ref[:, 0:HIDDEN] = wt_ref[...].T

  grid = (VOCAB + TBLK - 1) // TBLK
  return pl.pallas_call(
      body,
      grid=(grid,),
      in_specs=[pl.BlockSpec((HIDDEN, TBLK), lambda i: (0, i))],
      out_specs=pl.BlockSpec((TBLK, PAD), lambda i: (i, 0)),
      out_shape=jax.ShapeDtypeStruct((VOCAB, PAD), jnp.float32),
  )


def _sc_gather():
  mesh = plsc.VectorSubcoreMesh(
      core_axis_name="c", subcore_axis_name="s", num_cores=NC, num_subcores=NS
  )

  @functools.partial(
      pl.kernel,
      out_type=jax.ShapeDtypeStruct((HIST * HIDDEN, BATCH), jnp.float32),
      mesh=mesh,
      scratch_types=[
          pltpu.VMEM((HIST * TPW, CHUNK), jnp.int32),
          [pltpu.VMEM((CHUNK, PAD), jnp.float32) for _ in range(NBUF)],
          [pltpu.VMEM((HIDDEN, CHUNK), jnp.float32) for _ in range(NBUF)],
          [pltpu.SemaphoreType.DMA for _ in range(NBUF)],
          [pltpu.SemaphoreType.DMA for _ in range(NBUF)],
      ],
      compiler_params=pltpu.CompilerParams(
          use_tc_tiling_on_sc=True, needs_layout_passes=False
      ),
  )
  def k(idx_hbm, table_hbm, out_hbm, idx_v, bufs, bufts, g_sems, w_sems):
    wid = lax.axis_index("s") * NC + lax.axis_index("c")
    ibase = wid * IPW
    pltpu.sync_copy(idx_hbm.at[wid], idx_v)

    def gather(c, b):
      pltpu.async_copy(table_hbm.at[idx_v.at[c]], bufs[b], g_sems[b])

    def gather_wait(c, b):
      pltpu.make_async_copy(table_hbm.at[idx_v.at[c]], bufs[b], g_sems[b]).wait()

    def _dst(c):
      return out_hbm.at[
          pl.ds((c // TPW) * HIDDEN, HIDDEN),
          pl.ds(ibase + (c % TPW) * CHUNK, CHUNK),
      ]

    def wb(c, b):
      pltpu.async_copy(bufts[b], _dst(c), w_sems[b])

    def wb_wait(c, b):
      pltpu.make_async_copy(bufts[b], _dst(c), w_sems[b]).wait()

    def transpose(b):
      src, dst = bufs[b], bufts[b]

      @pl.loop(0, HIDDEN, unroll=8)
      def _(h):
        col = jnp.full((LANES,), h, jnp.int32)
        for m in range(CHUNK // LANES):
          row = m * LANES + lax.iota(jnp.int32, LANES)
          dst[h, pl.ds(m * LANES, LANES)] = plsc.load_gather(src, [row, col])

    nsteps = HIST * TPW
    for b in range(NBUF):
      gather(b, b)

    @pl.loop(0, nsteps, step=NBUF)
    def _(c):
      for b in range(NBUF):
        gather_wait(c + b, b)

        @pl.when(c + b >= NBUF)
        def _():
          wb_wait(c + b - NBUF, b)

        transpose(b)
        wb(c + b, b)

        @pl.when(c + b + NBUF < nsteps)
        def _():
          gather(c + b + NBUF, b)

    for b in range(NBUF):
      wb_wait(nsteps - NBUF + b, b)

  return k


_table_transpose_call = _tc_table_transpose()
_gather_call = _sc_gather()


@jax.jit
def kernel(v, weight):
  wt = jnp.swapaxes(weight, 0, 1)
  table = _table_transpose_call(wt)
  idx = (
      jnp.swapaxes(v, 0, 1)
      .reshape(HIST, NW, TPW, CHUNK)
      .transpose(1, 0, 2, 3)
      .reshape(NW, HIST * TPW, CHUNK)
  )
  out_t = _gather_call(idx, table)
  return out_t.reshape(HIST, HIDDEN, BATCH).transpose(2, 0, 1)


# speedup vs baseline: 1.9644x; 1.9644x over previous
import functools

import jax
import jax.numpy as jnp
from jax import lax
from jax.experimental import pallas as pl
from jax.experimental.pallas import tpu as pltpu
from jax.experimental.pallas import tpu_sc as plsc

VOCAB = 1000000
HIDDEN = 64
BATCH = 16384
HIST = 50
HALF = HIST // 2

NC = 2
NS = 16
NW = NC * NS

CHUNK = 128
IPW = BATCH // NW
TPW = IPW // CHUNK
NBUF = 5

PAD = 128
TBLK = 16384


def _tc_table_transpose():
  def body(wt_ref, out_ref):
    out_ref[:, 0:HIDDEN] = wt_ref[...].T

  grid = (VOCAB + TBLK - 1) // TBLK
  return pl.pallas_call(
      body,
      grid=(grid,),
      in_specs=[pl.BlockSpec((HIDDEN, TBLK), lambda i: (0, i))],
      out_specs=pl.BlockSpec((TBLK, PAD), lambda i: (i, 0)),
      out_shape=jax.ShapeDtypeStruct((VOCAB, PAD), jnp.float32),
  )


def _sc_gather_half():
  mesh = plsc.VectorSubcoreMesh(
      core_axis_name="c", subcore_axis_name="s", num_cores=NC, num_subcores=NS
  )
  nsteps = HALF * TPW

  @functools.partial(
      pl.kernel,
      out_type=jax.ShapeDtypeStruct((HALF, BATCH, PAD), jnp.float32),
      mesh=mesh,
      scratch_types=[
          pltpu.VMEM((nsteps, CHUNK), jnp.int32),
          [pltpu.VMEM((CHUNK, PAD), jnp.float32) for _ in range(NBUF)],
          [pltpu.SemaphoreType.DMA for _ in range(NBUF)],
          [pltpu.SemaphoreType.DMA for _ in range(NBUF)],
      ],
      compiler_params=pltpu.CompilerParams(use_tc_tiling_on_sc=True),
  )
  def k(idx_hbm, table_hbm, out_hbm, idx_v, bufs, g_sems, w_sems):
    wid = lax.axis_index("s") * NC + lax.axis_index("c")
    ibase = wid * IPW
    pltpu.sync_copy(idx_hbm.at[wid], idx_v)

    def gather(c, b):
      pltpu.async_copy(table_hbm.at[idx_v.at[c]], bufs[b], g_sems[b])

    def gather_wait(c, b):
      pltpu.make_async_copy(table_hbm.at[idx_v.at[c]], bufs[b], g_sems[b]).wait()

    def _dst(c):
      return out_hbm.at[c // TPW, pl.ds(ibase + (c % TPW) * CHUNK, CHUNK), :]

    def wb(c, b):
      pltpu.async_copy(bufs[b], _dst(c), w_sems[b])

    def wb_wait(c, b):
      pltpu.make_async_copy(bufs[b], _dst(c), w_sems[b]).wait()

    for b in range(NBUF):
      gather(b, b)

    @pl.loop(NBUF, nsteps, step=NBUF)
    def _(c):
      for b in range(NBUF):
        gather_wait(c - NBUF + b, b)
        wb(c - NBUF + b, b)
      for b in range(NBUF):
        wb_wait(c - NBUF + b, b)
        gather(c + b, b)

    for b in range(NBUF):
      gather_wait(nsteps - NBUF + b, b)
      wb(nsteps - NBUF + b, b)
    for b in range(NBUF):
      wb_wait(nsteps - NBUF + b, b)

  return k


IB = 8192


def _tc_out_transpose_half(j_off, aliased):
  if aliased:
    def body(_, in_ref, out_ref):
      out_ref[...] = in_ref[0, :, 0:HIDDEN].T
  else:
    def body(in_ref, out_ref):
      out_ref[...] = in_ref[0, :, 0:HIDDEN].T

  in_specs = [pl.BlockSpec((1, IB, PAD), lambda j, b: (j, b, 0))]
  if aliased:
    in_specs = [pl.BlockSpec(memory_space=pltpu.MemorySpace.HBM)] + in_specs
  return pl.pallas_call(
      body,
      grid=(HALF, BATCH // IB),
      in_specs=in_specs,
      out_specs=pl.BlockSpec((HIDDEN, IB), lambda j, b: (j + j_off, b)),
      out_shape=jax.ShapeDtypeStruct((HIST * HIDDEN, BATCH), jnp.float32),
      input_output_aliases={0: 0} if aliased else {},
  )


_table_transpose_call = _tc_table_transpose()
_gather_half_call = _sc_gather_half()
_out_xpose_a = _tc_out_transpose_half(0, aliased=False)
_out_xpose_b = _tc_out_transpose_half(HALF, aliased=True)


@jax.jit
def kernel(v, weight):
  wt = jnp.swapaxes(weight, 0, 1)
  table = _table_transpose_call(wt)
  idx = (
      jnp.swapaxes(v, 0, 1)
      .reshape(HIST, NW, TPW, CHUNK)
      .transpose(1, 0, 2, 3)
      .reshape(NW, HIST, TPW, CHUNK)
  )
  idx_a = idx[:, :HALF].reshape(NW, HALF * TPW, CHUNK)
  idx_b = idx[:, HALF:].reshape(NW, HALF * TPW, CHUNK)
  rows_a = _gather_half_call(idx_a, table)
  rows_b = _gather_half_call(idx_b, table)
  out_t = _out_xpose_a(rows_a)
  out_t = _out_xpose_b(out_t, rows_b)
  return out_t.reshape(HIST, HIDDEN, BATCH).transpose(2, 0, 1)


# speedup vs baseline: 1.9861x; 1.0111x over previous
import functools

import jax
import jax.numpy as jnp
from jax import lax
from jax.experimental import pallas as pl
from jax.experimental.pallas import tpu as pltpu
from jax.experimental.pallas import tpu_sc as plsc

VOCAB = 1000000
HIDDEN = 64
BATCH = 16384
HIST = 50

NC = 2
NS = 16
NW = NC * NS

CHUNK = 128
IPW = BATCH // NW
TPW = IPW // CHUNK
NBUF = 5

PAD = 128
TBLK = 16384


def _tc_table_transpose():
  def body(wt_ref, out_ref):
    out_ref[:, 0:HIDDEN] = wt_ref[...].T

  grid = (VOCAB + TBLK - 1) // TBLK
  return pl.pallas_call(
      body,
      grid=(grid,),
      in_specs=[pl.BlockSpec((HIDDEN, TBLK), lambda i: (0, i))],
      out_specs=pl.BlockSpec((TBLK, PAD), lambda i: (i, 0)),
      out_shape=jax.ShapeDtypeStruct((VOCAB, PAD), jnp.float32),
  )


def _sc_gather():
  mesh = plsc.VectorSubcoreMesh(
      core_axis_name="c", subcore_axis_name="s", num_cores=NC, num_subcores=NS
  )

  @functools.partial(
      pl.kernel,
      out_type=jax.ShapeDtypeStruct((HIST, BATCH, PAD), jnp.float32),
      mesh=mesh,
      scratch_types=[
          pltpu.VMEM((HIST * TPW, CHUNK), jnp.int32),
          [pltpu.VMEM((CHUNK, PAD), jnp.float32) for _ in range(NBUF)],
          [pltpu.SemaphoreType.DMA for _ in range(NBUF)],
          [pltpu.SemaphoreType.DMA for _ in range(NBUF)],
      ],
      compiler_params=pltpu.CompilerParams(use_tc_tiling_on_sc=True),
  )
  def k(idx_hbm, table_hbm, out_hbm, idx_v, bufs, g_sems, w_sems):
    wid = lax.axis_index("s") * NC + lax.axis_index("c")
    ibase = wid * IPW
    pltpu.sync_copy(idx_hbm.at[wid], idx_v)

    def gather(c, b):
      pltpu.async_copy(table_hbm.at[idx_v.at[c]], bufs[b], g_sems[b])

    def gather_wait(c, b):
      pltpu.make_async_copy(table_hbm.at[idx_v.at[c]], bufs[b], g_sems[b]).wait()

    def _dst(c):
      return out_hbm.at[c // TPW, pl.ds(ibase + (c % TPW) * CHUNK, CHUNK), :]

    def wb(c, b):
      pltpu.async_copy(bufs[b], _dst(c), w_sems[b])

    def wb_wait(c, b):
      pltpu.make_async_copy(bufs[b], _dst(c), w_sems[b]).wait()

    nsteps = HIST * TPW
    for b in range(NBUF):
      gather(b, b)

    @pl.loop(NBUF, nsteps, step=NBUF)
    def _(c):
      for b in range(NBUF):
        gather_wait(c - NBUF + b, b)
        wb(c - NBUF + b, b)
      for b in range(NBUF):
        wb_wait(c - NBUF + b, b)
        gather(c + b, b)

    for b in range(NBUF):
      gather_wait(nsteps - NBUF + b, b)
      wb(nsteps - NBUF + b, b)
    for b in range(NBUF):
      wb_wait(nsteps - NBUF + b, b)

  return k


IB = 8192
JB = 2


def _tc_out_transpose():
  def body(in_ref, out_ref):
    for q in range(JB):
      out_ref[q * HIDDEN:(q + 1) * HIDDEN, :] = in_ref[q, :, 0:HIDDEN].T

  return pl.pallas_call(
      body,
      grid=(HIST // JB, BATCH // IB),
      in_specs=[pl.BlockSpec((JB, IB, PAD), lambda j, b: (j, b, 0))],
      out_specs=pl.BlockSpec((JB * HIDDEN, IB), lambda j, b: (j, b)),
      out_shape=jax.ShapeDtypeStruct((HIST * HIDDEN, BATCH), jnp.float32),
  )


_table_transpose_call = _tc_table_transpose()
_gather_call = _sc_gather()
_out_transpose_call = _tc_out_transpose()


@jax.jit
def kernel(v, weight):
  wt = jnp.swapaxes(weight, 0, 1)
  table = _table_transpose_call(wt)
  idx = (
      jnp.swapaxes(v, 0, 1)
      .reshape(HIST, NW, TPW, CHUNK)
      .transpose(1, 0, 2, 3)
      .reshape(NW, HIST * TPW, CHUNK)
  )
  rows = _gather_call(idx, table)
  out_t = _out_transpose_call(rows)
  return out_t.reshape(HIST, HIDDEN, BATCH).transpose(2, 0, 1)
